# baseline (device time: 15914 ns/iter reference)
import jax
import jax.numpy as jnp
from jax import lax
from jax.experimental import pallas as pl
from jax.experimental.pallas import tpu as pltpu

N_DEV = 4
B, SQ, SKV, HQ, DH = 2, 256, 1024, 4, 64
S_LOC = SKV // N_DEV
DM = 512
BLK = 64
HD = HQ * DH
QR = SQ // N_DEV
ROWS = SQ + 16


def kernel(x, Wq, K_ext, V_ext, Wo):
    def body(x_ref, wq_ref, k_ref, v_ref, wo_ref, out_ref,
             sendbuf, statbuf, outstage,
             csend, crecv, ssend, srecv, osend, orecv):
        my = lax.axis_index("i")

        barrier = pltpu.get_barrier_semaphore()
        for off in (1, 2, 3):
            pl.semaphore_signal(barrier, inc=1,
                                device_id=((my + off) % N_DEV,),
                                device_id_type=pl.DeviceIdType.MESH)

        wq = (wq_ref[...] * 0.125).astype(jnp.bfloat16)
        wo = wo_ref[...].astype(jnp.bfloat16)

        def partial_attn(b):
            ctx = x_ref[b][:, :HD]
            stats_t = jnp.ones((SQ, HQ), jnp.float32)
            sendbuf[pl.ds(my, 1), b] = ctx.astype(jnp.bfloat16)[None]
            statbuf[pl.ds(my, 1), b] = stats_t[None]

        def fire_p1(b):
            rdmas = []
            for idx, off in enumerate((1, 2, 3)):
                t = (my + off) % N_DEV
                crdma = pltpu.make_async_remote_copy(
                    src_ref=sendbuf.at[my, b, pl.ds(t * QR, QR)],
                    dst_ref=sendbuf.at[my, b, pl.ds(t * QR, QR)],
                    send_sem=csend.at[b * 3 + idx],
                    recv_sem=crecv.at[b * 3 + idx],
                    device_id=(t,), device_id_type=pl.DeviceIdType.MESH)
                srdma = pltpu.make_async_remote_copy(
                    src_ref=statbuf.at[my, b, pl.ds(t * QR, QR)],
                    dst_ref=statbuf.at[my, b, pl.ds(t * QR, QR)],
                    send_sem=ssend.at[b * 3 + idx],
                    recv_sem=srecv.at[b * 3 + idx],
                    device_id=(t,), device_id_type=pl.DeviceIdType.MESH)
                crdma.start()
                srdma.start()
                rdmas.extend((crdma, srdma))
            return rdmas

        def combine(b, rdmas):
            num = sendbuf[pl.ds(my, 1), b,
                          pl.ds(my * QR, QR)][0].astype(jnp.float32)
            den = statbuf[pl.ds(my, 1), b,
                          pl.ds(my * QR, QR)][0]
            for idx in range(N_DEV - 1):
                rdmas[idx * 2].wait_recv()
                rdmas[idx * 2 + 1].wait_recv()
                slot = (my + N_DEV - 1 - idx) % N_DEV
                arr_c = sendbuf[pl.ds(slot, 1), b,
                                pl.ds(my * QR, QR)]
                arr_s = statbuf[pl.ds(slot, 1), b,
                                pl.ds(my * QR, QR)]
                num = num + arr_c[0].astype(jnp.float32)
                den = den + arr_s[0]
            d = jnp.broadcast_to(den[:, :, None], (QR, HQ, DH))
            outq = jnp.dot((num / d.reshape(QR, HD)).astype(jnp.bfloat16),
                           wo, preferred_element_type=jnp.float32)
            outstage[pl.ds(my, 1), b] = outq.astype(jnp.bfloat16)[None]
            p2 = []
            for idx, off in enumerate((1, 2, 3)):
                rdma = pltpu.make_async_remote_copy(
                    src_ref=outstage.at[my, b], dst_ref=outstage.at[my, b],
                    send_sem=osend.at[b * 3 + idx],
                    recv_sem=orecv.at[b * 3 + idx],
                    device_id=((my + off) % N_DEV,),
                    device_id_type=pl.DeviceIdType.MESH)
                rdma.start()
                p2.append(rdma)
            out_ref[b, pl.ds(my * QR, QR)] = outq
            return p2

        def drain(b, p2):
            for idx in range(N_DEV - 1):
                p2[idx].wait_recv()
                slot = (my + N_DEV - 1 - idx) % N_DEV
                arr = outstage[pl.ds(slot, 1), b]
                out_ref[b, pl.ds(slot * QR, QR)] = arr[0].astype(jnp.float32)

        partial_attn(0)
        pl.semaphore_wait(barrier, N_DEV - 1)
        p1_0 = fire_p1(0)
        partial_attn(1)
        p1_1 = fire_p1(1)
        p2_0 = combine(0, p1_0)
        p2_1 = combine(1, p1_1)
        drain(0, p2_0)
        drain(1, p2_1)

        for rdmas in (p1_0, p1_1):
            for rdma in rdmas:
                rdma.wait_send()
        for rdmas in (p2_0, p2_1):
            for rdma in rdmas:
                rdma.wait_send()

    return pl.pallas_call(
        body,
        out_shape=jax.ShapeDtypeStruct((B, SQ, DM), jnp.float32),
        in_specs=[pl.BlockSpec(memory_space=pltpu.VMEM)] * 5,
        out_specs=pl.BlockSpec(memory_space=pltpu.VMEM),
        scratch_shapes=[
            pltpu.VMEM((N_DEV, B, SQ, HD), jnp.bfloat16),
            pltpu.VMEM((N_DEV, B, SQ, HQ), jnp.float32),
            pltpu.VMEM((N_DEV, B, QR, DM), jnp.bfloat16),
            pltpu.SemaphoreType.DMA((B * 3,)),
            pltpu.SemaphoreType.DMA((B * 3,)),
            pltpu.SemaphoreType.DMA((B * 3,)),
            pltpu.SemaphoreType.DMA((B * 3,)),
            pltpu.SemaphoreType.DMA((B * 3,)),
            pltpu.SemaphoreType.DMA((B * 3,)),
        ],
        compiler_params=pltpu.CompilerParams(collective_id=0),
    )(x, Wq, K_ext, V_ext, Wo)


# device time: 15231 ns/iter; 1.0448x vs baseline; 1.0448x over previous
import jax
import jax.numpy as jnp
from jax import lax
from jax.experimental import pallas as pl
from jax.experimental.pallas import tpu as pltpu

N_DEV = 4
B, SQ, SKV, HQ, DH = 2, 256, 1024, 4, 64
S_LOC = SKV // N_DEV
DM = 512
BLK = 64
HD = HQ * DH
QR = SQ // N_DEV
PR = QR + 16


def kernel(x, Wq, K_ext, V_ext, Wo):
    def body(x_ref, wq_ref, k_ref, v_ref, wo_ref, out_ref,
             sendbuf, outstage, csend, crecv, osend, orecv):
        my = lax.axis_index("i")

        barrier = pltpu.get_barrier_semaphore()
        for off in (1, 2, 3):
            pl.semaphore_signal(barrier, inc=1,
                                device_id=((my + off) % N_DEV,),
                                device_id_type=pl.DeviceIdType.MESH)

        wq = (wq_ref[...] * 0.125).astype(jnp.bfloat16)
        wo = wo_ref[...].astype(jnp.bfloat16)

        def partial_attn(b):
            xb = x_ref[b].astype(jnp.bfloat16)
            q = jnp.dot(xb, wq, preferred_element_type=jnp.float32)
            kb = k_ref[b].reshape(S_LOC, HD).astype(jnp.bfloat16)
            vb = v_ref[b].reshape(S_LOC, HD).astype(jnp.bfloat16)
            ctx_h = []
            s_h = []
            for h in range(HQ):
                q3 = (q[:, h * DH:(h + 1) * DH].astype(jnp.bfloat16)
                      .reshape(SQ // BLK, BLK, DH))
                k3 = kb[:, h * DH:(h + 1) * DH].reshape(S_LOC // BLK, BLK, DH)
                v3 = vb[:, h * DH:(h + 1) * DH].reshape(S_LOC // BLK, BLK, DH)
                s = lax.dot_general(
                    q3, k3, (((2,), (2,)), ((0,), (0,))),
                    preferred_element_type=jnp.float32)
                e = jnp.exp(s)
                s_h.append(jnp.sum(e, axis=-1).reshape(SQ))
                ctx_h.append(lax.dot_general(
                    e.astype(jnp.bfloat16), v3,
                    (((2,), (1,)), ((0,), (0,))),
                    preferred_element_type=jnp.float32).reshape(SQ, DH))
            ctx = jnp.concatenate(ctx_h, axis=1)
            stats_hq = jnp.stack(s_h, axis=0)
            ctx4 = ctx.astype(jnp.bfloat16).reshape(N_DEV, QR, HD)
            sendbuf[pl.ds(my, 1), b, pl.ds(0, N_DEV), pl.ds(0, QR)] = (
                ctx4[None])
            sq = stats_hq.astype(jnp.bfloat16)
            for r in range(N_DEV):
                sendbuf[pl.ds(my, 1), b, pl.ds(r, 1),
                        pl.ds(QR, HQ), pl.ds(0, BLK)] = (
                    sq[:, r * BLK:(r + 1) * BLK][None, None])

        def fire_p1(b):
            rdmas = []
            for idx, off in enumerate((1, 2, 3)):
                t = (my + off) % N_DEV
                rdma = pltpu.make_async_remote_copy(
                    src_ref=sendbuf.at[my, b, pl.ds(t, 1)],
                    dst_ref=sendbuf.at[my, b, pl.ds(t, 1)],
                    send_sem=csend.at[b * 3 + idx],
                    recv_sem=crecv.at[b * 3 + idx],
                    device_id=(t,), device_id_type=pl.DeviceIdType.MESH)
                rdma.start()
                rdmas.append(rdma)
            return rdmas

        def combine(b, rdmas):
            own = sendbuf[pl.ds(my, 1), b, pl.ds(my, 1)]
            num = own[0, 0, :QR, :].astype(jnp.float32)
            den = own[0, 0, QR:QR + HQ, :BLK].astype(jnp.float32)
            for idx in range(N_DEV - 1):
                rdmas[idx].wait_recv()
                slot = (my + N_DEV - 1 - idx) % N_DEV
                arr = sendbuf[pl.ds(slot, 1), b, pl.ds(my, 1)]
                num = num + arr[0, 0, :QR, :].astype(jnp.float32)
                den = den + arr[0, 0, QR:QR + HQ, :BLK].astype(jnp.float32)
            d = jnp.broadcast_to(den.T[:, :, None], (QR, HQ, DH))
            outq = jnp.dot((num / d.reshape(QR, HD)).astype(jnp.bfloat16),
                           wo, preferred_element_type=jnp.float32)
            outstage[pl.ds(my, 1), b] = outq.astype(jnp.bfloat16)[None]
            p2 = []
            for idx, off in enumerate((1, 2, 3)):
                rdma = pltpu.make_async_remote_copy(
                    src_ref=outstage.at[my, b], dst_ref=outstage.at[my, b],
                    send_sem=osend.at[b * 3 + idx],
                    recv_sem=orecv.at[b * 3 + idx],
                    device_id=((my + off) % N_DEV,),
                    device_id_type=pl.DeviceIdType.MESH)
                rdma.start()
                p2.append(rdma)
            out_ref[b, pl.ds(my * QR, QR)] = outq
            return p2

        def drain(b, p2):
            for idx in range(N_DEV - 1):
                p2[idx].wait_recv()
                slot = (my + N_DEV - 1 - idx) % N_DEV
                arr = outstage[pl.ds(slot, 1), b]
                out_ref[b, pl.ds(slot * QR, QR)] = arr[0].astype(jnp.float32)

        partial_attn(0)
        pl.semaphore_wait(barrier, N_DEV - 1)
        p1_0 = fire_p1(0)
        partial_attn(1)
        p1_1 = fire_p1(1)
        p2_0 = combine(0, p1_0)
        p2_1 = combine(1, p1_1)
        drain(0, p2_0)
        drain(1, p2_1)

        for rdmas in (p1_0, p1_1, p2_0, p2_1):
            for rdma in rdmas:
                rdma.wait_send()

    return pl.pallas_call(
        body,
        out_shape=jax.ShapeDtypeStruct((B, SQ, DM), jnp.float32),
        in_specs=[pl.BlockSpec(memory_space=pltpu.VMEM)] * 5,
        out_specs=pl.BlockSpec(memory_space=pltpu.VMEM),
        scratch_shapes=[
            pltpu.VMEM((N_DEV, B, N_DEV, PR, HD), jnp.bfloat16),
            pltpu.VMEM((N_DEV, B, QR, DM), jnp.bfloat16),
            pltpu.SemaphoreType.DMA((B * 3,)),
            pltpu.SemaphoreType.DMA((B * 3,)),
            pltpu.SemaphoreType.DMA((B * 3,)),
            pltpu.SemaphoreType.DMA((B * 3,)),
        ],
        compiler_params=pltpu.CompilerParams(collective_id=0),
    )(x, Wq, K_ext, V_ext, Wo)


# device time: 15064 ns/iter; 1.0564x vs baseline; 1.0111x over previous
import jax
import jax.numpy as jnp
from jax import lax
from jax.experimental import pallas as pl
from jax.experimental.pallas import tpu as pltpu

N_DEV = 4
B, SQ, SKV, HQ, DH = 2, 256, 1024, 4, 64
S_LOC = SKV // N_DEV
DM = 512
BLK = 64
HD = HQ * DH
QR = SQ // N_DEV
PR = QR + 16


def kernel(x, Wq, K_ext, V_ext, Wo):
    def body(x_ref, wq_ref, k_ref, v_ref, wo_ref, out_ref,
             sendbuf, csend, crecv, osend, orecv):
        my = lax.axis_index("i")

        barrier = pltpu.get_barrier_semaphore()
        for off in (1, 2, 3):
            pl.semaphore_signal(barrier, inc=1,
                                device_id=((my + off) % N_DEV,),
                                device_id_type=pl.DeviceIdType.MESH)

        wq = (wq_ref[...] * 0.125).astype(jnp.bfloat16)
        wo = wo_ref[...].astype(jnp.bfloat16)

        def partial_attn(b):
            xb = x_ref[b].astype(jnp.bfloat16)
            q = jnp.dot(xb, wq, preferred_element_type=jnp.float32)
            kb = k_ref[b].reshape(S_LOC, HD).astype(jnp.bfloat16)
            vb = v_ref[b].reshape(S_LOC, HD).astype(jnp.bfloat16)
            ctx_h = []
            s_h = []
            for h in range(HQ):
                q3 = (q[:, h * DH:(h + 1) * DH].astype(jnp.bfloat16)
                      .reshape(SQ // BLK, BLK, DH))
                k3 = kb[:, h * DH:(h + 1) * DH].reshape(S_LOC // BLK, BLK, DH)
                v3 = vb[:, h * DH:(h + 1) * DH].reshape(S_LOC // BLK, BLK, DH)
                s = lax.dot_general(
                    q3, k3, (((2,), (2,)), ((0,), (0,))),
                    preferred_element_type=jnp.float32)
                e = jnp.exp(s)
                s_h.append(jnp.sum(e, axis=-1).reshape(SQ))
                ctx_h.append(lax.dot_general(
                    e.astype(jnp.bfloat16), v3,
                    (((2,), (1,)), ((0,), (0,))),
                    preferred_element_type=jnp.float32).reshape(SQ, DH))
            ctx = jnp.concatenate(ctx_h, axis=1)
            stats_hq = jnp.stack(s_h, axis=0)
            ctx4 = ctx.astype(jnp.bfloat16).reshape(N_DEV, QR, HD)
            sendbuf[pl.ds(my, 1), b, pl.ds(0, N_DEV), pl.ds(0, QR)] = (
                ctx4[None])
            sq = stats_hq.astype(jnp.bfloat16)
            for r in range(N_DEV):
                sendbuf[pl.ds(my, 1), b, pl.ds(r, 1),
                        pl.ds(QR, HQ), pl.ds(0, BLK)] = (
                    sq[:, r * BLK:(r + 1) * BLK][None, None])

        def fire_p1(b):
            rdmas = []
            for idx, off in enumerate((1, 2, 3)):
                t = (my + off) % N_DEV
                rdma = pltpu.make_async_remote_copy(
                    src_ref=sendbuf.at[my, b, pl.ds(t, 1)],
                    dst_ref=sendbuf.at[my, b, pl.ds(t, 1)],
                    send_sem=csend.at[b * 3 + idx],
                    recv_sem=crecv.at[b * 3 + idx],
                    device_id=(t,), device_id_type=pl.DeviceIdType.MESH)
                rdma.start()
                rdmas.append(rdma)
            return rdmas

        def combine(b, rdmas):
            own = sendbuf[pl.ds(my, 1), b, pl.ds(my, 1)]
            num = own[0, 0, :QR, :].astype(jnp.float32)
            den = own[0, 0, QR:QR + HQ, :BLK].astype(jnp.float32)
            for idx in range(N_DEV - 1):
                rdmas[idx].wait_recv()
                slot = (my + N_DEV - 1 - idx) % N_DEV
                arr = sendbuf[pl.ds(slot, 1), b, pl.ds(my, 1)]
                num = num + arr[0, 0, :QR, :].astype(jnp.float32)
                den = den + arr[0, 0, QR:QR + HQ, :BLK].astype(jnp.float32)
            d = jnp.broadcast_to(den.T[:, :, None], (QR, HQ, DH))
            outq = jnp.dot((num / d.reshape(QR, HD)).astype(jnp.bfloat16),
                           wo, preferred_element_type=jnp.float32)
            out_ref[b, pl.ds(my * QR, QR)] = outq.astype(jnp.bfloat16)
            p2 = []
            for idx, off in enumerate((1, 2, 3)):
                rdma = pltpu.make_async_remote_copy(
                    src_ref=out_ref.at[b, pl.ds(my * QR, QR)],
                    dst_ref=out_ref.at[b, pl.ds(my * QR, QR)],
                    send_sem=osend.at[b * 3 + idx],
                    recv_sem=orecv.at[b * 3 + idx],
                    device_id=((my + off) % N_DEV,),
                    device_id_type=pl.DeviceIdType.MESH)
                rdma.start()
                p2.append(rdma)
            return p2

        def drain(b, p2):
            for idx in range(N_DEV - 1):
                p2[idx].wait_recv()

        partial_attn(0)
        pl.semaphore_wait(barrier, N_DEV - 1)
        p1_0 = fire_p1(0)
        partial_attn(1)
        p1_1 = fire_p1(1)
        p2_0 = combine(0, p1_0)
        p2_1 = combine(1, p1_1)
        drain(0, p2_0)
        drain(1, p2_1)

        for rdmas in (p1_0, p1_1, p2_0, p2_1):
            for rdma in rdmas:
                rdma.wait_send()

    return pl.pallas_call(
        body,
        out_shape=jax.ShapeDtypeStruct((B, SQ, DM), jnp.bfloat16),
        in_specs=[pl.BlockSpec(memory_space=pltpu.VMEM)] * 5,
        out_specs=pl.BlockSpec(memory_space=pltpu.VMEM),
        scratch_shapes=[
            pltpu.VMEM((N_DEV, B, N_DEV, PR, HD), jnp.bfloat16),
            pltpu.SemaphoreType.DMA((B * 3,)),
            pltpu.SemaphoreType.DMA((B * 3,)),
            pltpu.SemaphoreType.DMA((B * 3,)),
            pltpu.SemaphoreType.DMA((B * 3,)),
        ],
        compiler_params=pltpu.CompilerParams(collective_id=0),
    )(x, Wq, K_ext, V_ext, Wo)


# device time: 15020 ns/iter; 1.0595x vs baseline; 1.0029x over previous
import jax
import jax.numpy as jnp
from jax import lax
from jax.experimental import pallas as pl
from jax.experimental.pallas import tpu as pltpu

N_DEV = 4
B, SQ, SKV, HQ, DH = 2, 256, 1024, 4, 64
S_LOC = SKV // N_DEV
DM = 512
BLK = 64
HD = HQ * DH
QR = SQ // N_DEV
PR = QR + 16


def kernel(x, Wq, K_ext, V_ext, Wo):
    def body(x_ref, wq_ref, k_ref, v_ref, wo_ref, out_ref,
             sendbuf, csend, crecv, osend, orecv):
        my = lax.axis_index("i")

        barrier = pltpu.get_barrier_semaphore()
        for off in (1, 2, 3):
            pl.semaphore_signal(barrier, inc=1,
                                device_id=((my + off) % N_DEV,),
                                device_id_type=pl.DeviceIdType.MESH)

        wq = (wq_ref[...] * 0.125).astype(jnp.bfloat16)
        wo = wo_ref[...].astype(jnp.bfloat16)

        def partial_attn(b):
            xb = x_ref[b].astype(jnp.bfloat16)
            q = jnp.dot(xb, wq, preferred_element_type=jnp.float32)
            kb = k_ref[b].reshape(S_LOC, HD).astype(jnp.bfloat16)
            vb = v_ref[b].reshape(S_LOC, HD).astype(jnp.bfloat16)
            ctx_h = []
            s_h = []
            for h in range(HQ):
                q3 = (q[:, h * DH:(h + 1) * DH].astype(jnp.bfloat16)
                      .reshape(SQ // BLK, BLK, DH))
                k3 = kb[:, h * DH:(h + 1) * DH].reshape(S_LOC // BLK, BLK, DH)
                v3 = vb[:, h * DH:(h + 1) * DH].reshape(S_LOC // BLK, BLK, DH)
                s = lax.dot_general(
                    q3, k3, (((2,), (2,)), ((0,), (0,))),
                    preferred_element_type=jnp.float32)
                e = jnp.exp(s)
                s_h.append(jnp.sum(e, axis=-1).reshape(SQ))
                ctx_h.append(lax.dot_general(
                    e.astype(jnp.bfloat16), v3,
                    (((2,), (1,)), ((0,), (0,))),
                    preferred_element_type=jnp.float32).reshape(SQ, DH))
            ctx = jnp.concatenate(ctx_h, axis=1)
            stats_hq = jnp.stack(s_h, axis=0)
            ctx4 = ctx.astype(jnp.bfloat16).reshape(N_DEV, QR, HD)
            sendbuf[pl.ds(my, 1), b, pl.ds(0, N_DEV), pl.ds(0, QR)] = (
                ctx4[None])
            sq = stats_hq.astype(jnp.bfloat16)
            for r in range(N_DEV):
                sendbuf[pl.ds(my, 1), b, pl.ds(r, 1),
                        pl.ds(QR, HQ), pl.ds(0, BLK)] = (
                    sq[:, r * BLK:(r + 1) * BLK][None, None])

        def fire_p1(b):
            rdmas = []
            for idx, off in enumerate((1, 2, 3)):
                t = (my + off) % N_DEV
                rdma = pltpu.make_async_remote_copy(
                    src_ref=sendbuf.at[my, b, pl.ds(t, 1), pl.ds(0, QR + 8)],
                    dst_ref=sendbuf.at[my, b, pl.ds(t, 1), pl.ds(0, QR + 8)],
                    send_sem=csend.at[b * 3 + idx],
                    recv_sem=crecv.at[b * 3 + idx],
                    device_id=(t,), device_id_type=pl.DeviceIdType.MESH)
                rdma.start()
                rdmas.append(rdma)
            return rdmas

        def combine(b, rdmas):
            own = sendbuf[pl.ds(my, 1), b, pl.ds(my, 1)]
            num = own[0, 0, :QR, :].astype(jnp.float32)
            den = own[0, 0, QR:QR + HQ, :BLK].astype(jnp.float32)
            for idx in (0, 2, 1):
                rdmas[idx].wait_recv()
                slot = (my + N_DEV - 1 - idx) % N_DEV
                arr = sendbuf[pl.ds(slot, 1), b, pl.ds(my, 1)]
                num = num + arr[0, 0, :QR, :].astype(jnp.float32)
                den = den + arr[0, 0, QR:QR + HQ, :BLK].astype(jnp.float32)
            d = jnp.broadcast_to(den.T[:, :, None], (QR, HQ, DH))
            outq = jnp.dot((num / d.reshape(QR, HD)).astype(jnp.bfloat16),
                           wo, preferred_element_type=jnp.float32)
            out_ref[b, pl.ds(my * QR, QR)] = outq.astype(jnp.bfloat16)
            p2 = []
            for idx, off in enumerate((1, 2, 3)):
                rdma = pltpu.make_async_remote_copy(
                    src_ref=out_ref.at[b, pl.ds(my * QR, QR)],
                    dst_ref=out_ref.at[b, pl.ds(my * QR, QR)],
                    send_sem=osend.at[b * 3 + idx],
                    recv_sem=orecv.at[b * 3 + idx],
                    device_id=((my + off) % N_DEV,),
                    device_id_type=pl.DeviceIdType.MESH)
                rdma.start()
                p2.append(rdma)
            return p2

        def drain(b, p2):
            for idx in range(N_DEV - 1):
                p2[idx].wait_recv()

        partial_attn(0)
        pl.semaphore_wait(barrier, N_DEV - 1)
        p1_0 = fire_p1(0)
        partial_attn(1)
        p1_1 = fire_p1(1)
        p2_0 = combine(0, p1_0)
        p2_1 = combine(1, p1_1)
        drain(0, p2_0)
        drain(1, p2_1)

        for rdmas in (p1_0, p1_1, p2_0, p2_1):
            for rdma in rdmas:
                rdma.wait_send()

    return pl.pallas_call(
        body,
        out_shape=jax.ShapeDtypeStruct((B, SQ, DM), jnp.bfloat16),
        in_specs=[pl.BlockSpec(memory_space=pltpu.VMEM)] * 5,
        out_specs=pl.BlockSpec(memory_space=pltpu.VMEM),
        scratch_shapes=[
            pltpu.VMEM((N_DEV, B, N_DEV, PR, HD), jnp.bfloat16),
            pltpu.SemaphoreType.DMA((B * 3,)),
            pltpu.SemaphoreType.DMA((B * 3,)),
            pltpu.SemaphoreType.DMA((B * 3,)),
            pltpu.SemaphoreType.DMA((B * 3,)),
        ],
        compiler_params=pltpu.CompilerParams(collective_id=0),
    )(x, Wq, K_ext, V_ext, Wo)


# device time: 14917 ns/iter; 1.0668x vs baseline; 1.0069x over previous
import jax
import jax.numpy as jnp
from jax import lax
from jax.experimental import pallas as pl
from jax.experimental.pallas import tpu as pltpu

N_DEV = 4
B, SQ, SKV, HQ, DH = 2, 256, 1024, 4, 64
S_LOC = SKV // N_DEV
DM = 512
BLK = 64
HD = HQ * DH
QR = SQ // N_DEV
PR = QR + 16


def kernel(x, Wq, K_ext, V_ext, Wo):
    def body(x_ref, wq_ref, k_ref, v_ref, wo_ref, out_ref,
             sendbuf, csend, crecv, osend, orecv):
        my = lax.axis_index("i")

        barrier = pltpu.get_barrier_semaphore()
        for off in (1, 2, 3):
            pl.semaphore_signal(barrier, inc=1,
                                device_id=((my + off) % N_DEV,),
                                device_id_type=pl.DeviceIdType.MESH)

        wq = (wq_ref[...] * 0.125).astype(jnp.bfloat16)
        wo = wo_ref[...].astype(jnp.bfloat16)

        def partial_attn(b):
            xb = x_ref[b].astype(jnp.bfloat16)
            q = jnp.dot(xb, wq, preferred_element_type=jnp.float32)
            kb = k_ref[b].reshape(S_LOC, HD).astype(jnp.bfloat16)
            vb = v_ref[b].reshape(S_LOC, HD).astype(jnp.bfloat16)
            ctx_h = []
            s_h = []
            for h in range(HQ):
                q3 = (q[:, h * DH:(h + 1) * DH].astype(jnp.bfloat16)
                      .reshape(SQ // BLK, BLK, DH))
                k3 = kb[:, h * DH:(h + 1) * DH].reshape(S_LOC // BLK, BLK, DH)
                v3 = vb[:, h * DH:(h + 1) * DH].reshape(S_LOC // BLK, BLK, DH)
                s = lax.dot_general(
                    q3, k3, (((2,), (2,)), ((0,), (0,))),
                    preferred_element_type=jnp.float32)
                e = jnp.exp(s)
                s_h.append(jnp.sum(e, axis=-1).reshape(SQ))
                ctx_h.append(lax.dot_general(
                    e.astype(jnp.bfloat16), v3,
                    (((2,), (1,)), ((0,), (0,))),
                    preferred_element_type=jnp.float32).reshape(SQ, DH))
            ctx = jnp.concatenate(ctx_h, axis=1)
            stats_hq = jnp.stack(s_h, axis=0)
            ctx4 = ctx.astype(jnp.bfloat16).reshape(N_DEV, QR, HD)
            sendbuf[pl.ds(my, 1), b, pl.ds(0, N_DEV), pl.ds(0, QR)] = (
                ctx4[None])
            sq = stats_hq.astype(jnp.bfloat16)
            for r in range(N_DEV):
                sendbuf[pl.ds(my, 1), b, pl.ds(r, 1),
                        pl.ds(QR, HQ), pl.ds(0, BLK)] = (
                    sq[:, r * BLK:(r + 1) * BLK][None, None])

        def fire_p1(b):
            rdmas = []
            for idx, off in enumerate((1, 2, 3)):
                t = (my + off) % N_DEV
                rdma = pltpu.make_async_remote_copy(
                    src_ref=sendbuf.at[my, b, pl.ds(t, 1), pl.ds(0, QR + 8)],
                    dst_ref=sendbuf.at[my, b, pl.ds(t, 1), pl.ds(0, QR + 8)],
                    send_sem=csend.at[b * 3 + idx],
                    recv_sem=crecv.at[b * 3 + idx],
                    device_id=(t,), device_id_type=pl.DeviceIdType.MESH)
                rdmas.append(rdma)
            for idx in (1, 0, 2):
                rdmas[idx].start()
            return rdmas

        def combine(b, rdmas):
            own = sendbuf[pl.ds(my, 1), b, pl.ds(my, 1)]
            num = own[0, 0, :QR, :].astype(jnp.float32)
            den = own[0, 0, QR:QR + HQ, :BLK].astype(jnp.float32)
            for idx in (0, 2, 1):
                rdmas[idx].wait_recv()
                slot = (my + N_DEV - 1 - idx) % N_DEV
                arr = sendbuf[pl.ds(slot, 1), b, pl.ds(my, 1)]
                num = num + arr[0, 0, :QR, :].astype(jnp.float32)
                den = den + arr[0, 0, QR:QR + HQ, :BLK].astype(jnp.float32)
            d = jnp.broadcast_to(den.T[:, :, None], (QR, HQ, DH))
            outq = jnp.dot((num / d.reshape(QR, HD)).astype(jnp.bfloat16),
                           wo, preferred_element_type=jnp.float32)
            out_ref[b, pl.ds(my * QR, QR)] = outq.astype(jnp.bfloat16)
            p2 = []
            for idx, off in enumerate((1, 2, 3)):
                rdma = pltpu.make_async_remote_copy(
                    src_ref=out_ref.at[b, pl.ds(my * QR, QR)],
                    dst_ref=out_ref.at[b, pl.ds(my * QR, QR)],
                    send_sem=osend.at[b * 3 + idx],
                    recv_sem=orecv.at[b * 3 + idx],
                    device_id=((my + off) % N_DEV,),
                    device_id_type=pl.DeviceIdType.MESH)
                p2.append(rdma)
            for idx in (1, 0, 2):
                p2[idx].start()
            return p2

        def drain(b, p2):
            for idx in range(N_DEV - 1):
                p2[idx].wait_recv()

        partial_attn(0)
        pl.semaphore_wait(barrier, N_DEV - 1)
        p1_0 = fire_p1(0)
        partial_attn(1)
        p1_1 = fire_p1(1)
        p2_0 = combine(0, p1_0)
        p2_1 = combine(1, p1_1)
        drain(0, p2_0)
        drain(1, p2_1)

        for rdmas in (p1_0, p1_1, p2_0, p2_1):
            for rdma in rdmas:
                rdma.wait_send()

    return pl.pallas_call(
        body,
        out_shape=jax.ShapeDtypeStruct((B, SQ, DM), jnp.bfloat16),
        in_specs=[pl.BlockSpec(memory_space=pltpu.VMEM)] * 5,
        out_specs=pl.BlockSpec(memory_space=pltpu.VMEM),
        scratch_shapes=[
            pltpu.VMEM((N_DEV, B, N_DEV, PR, HD), jnp.bfloat16),
            pltpu.SemaphoreType.DMA((B * 3,)),
            pltpu.SemaphoreType.DMA((B * 3,)),
            pltpu.SemaphoreType.DMA((B * 3,)),
            pltpu.SemaphoreType.DMA((B * 3,)),
        ],
        compiler_params=pltpu.CompilerParams(collective_id=0),
    )(x, Wq, K_ext, V_ext, Wo)
